# baseline (speedup 1.0000x reference)
import functools
import jax
import jax.numpy as jnp
from jax import lax
from jax.experimental import pallas as pl
from jax.experimental.pallas import tpu as pltpu
from jax.experimental.pallas import tpu_sc as plsc

_H = 32
_W = 32
_D = 256
_B = 16
_LANES = 16


def _sc_pos_kernel(row_hbm, col_hbm, out_hbm, slab, rowbuf, sem):
    nc = 2
    hc = 2
    wid = lax.axis_index("s") * nc + lax.axis_index("c")
    hgrp = wid % (_H // hc)
    half = wid // (_H // hc)
    nb = _B // 2
    for r in range(hc):
        pltpu.sync_copy(col_hbm, slab.at[r, :, pl.ds(0, _D)])
    pltpu.sync_copy(row_hbm.at[pl.ds(hgrp * hc, hc)], rowbuf)
    for r in range(hc):
        for c in range(_D // _LANES):
            v = rowbuf[r, pl.ds(c * _LANES, _LANES)]
            for i in range(_W):
                slab[r, i, pl.ds(_D + c * _LANES, _LANES)] = v
    copies = [
        pltpu.make_async_copy(
            slab, out_hbm.at[half * nb + k, pl.ds(hgrp * hc, hc)], sem.at[k]
        )
        for k in range(nb)
    ]
    for cp in copies:
        cp.start()
    for cp in copies:
        cp.wait()


@functools.partial(jax.jit, static_argnums=())
def _sc_call(row_embed, col_embed):
    mesh = plsc.VectorSubcoreMesh(core_axis_name="c", subcore_axis_name="s")
    kern = functools.partial(
        pl.kernel,
        mesh=mesh,
        out_type=jax.ShapeDtypeStruct((_B, _H, _W, 2 * _D), jnp.float32),
        scratch_types=[
            pltpu.VMEM((2, _W, 2 * _D), jnp.float32),
            pltpu.VMEM((2, _D), jnp.float32),
            pltpu.SemaphoreType.DMA((_B // 2,)),
        ],
    )(_sc_pos_kernel)
    return kern(row_embed, col_embed)


def kernel(x, row_embed, col_embed):
    b = x.shape[0]
    h, w = x.shape[-3], x.shape[-2]
    d = row_embed.shape[-1]
    out = _sc_call(row_embed, col_embed)
    return out.reshape(b, h * w, 2 * d)
